# baseline (speedup 1.0000x reference)
import jax
import jax.numpy as jnp
from jax.experimental import pallas as pl
from jax.experimental.pallas import tpu as pltpu

B, H, KV, DH = 16, 16, 4096, 128
BH = B * H
NCHUNK = 16


def _dma_copy_patch_kernel(idx_ref, cur_ref, cache_ref, out_ref, sems, psem):
    rows = cache_ref.shape[0] // NCHUNK
    copies = [
        pltpu.make_async_copy(
            cache_ref.at[pl.ds(c * rows, rows)],
            out_ref.at[pl.ds(c * rows, rows)],
            sems.at[c],
        )
        for c in range(NCHUNK)
    ]
    for cp in copies:
        cp.start()
    for cp in copies:
        cp.wait()
    slot = idx_ref[0] - 1
    patch = pltpu.make_async_copy(
        cur_ref,
        out_ref.at[:, pl.ds(slot, 1), :],
        psem,
    )
    patch.start()
    patch.wait()


def kernel(cur, dim, idx, cache):
    del dim
    cache3 = cache.reshape(BH, KV, DH)
    cur3 = cur.reshape(BH, 1, DH)

    out = pl.pallas_call(
        _dma_copy_patch_kernel,
        in_specs=[
            pl.BlockSpec(memory_space=pltpu.SMEM),
            pl.BlockSpec(memory_space=pl.ANY),
            pl.BlockSpec(memory_space=pl.ANY),
        ],
        out_specs=pl.BlockSpec(memory_space=pl.ANY),
        out_shape=jax.ShapeDtypeStruct((BH, KV, DH), cache.dtype),
        scratch_shapes=[
            pltpu.SemaphoreType.DMA((NCHUNK,)),
            pltpu.SemaphoreType.DMA,
        ],
    )(idx, cur3, cache3)
    return out.reshape(B, H, KV, DH)


# speedup vs baseline: 48.7173x; 48.7173x over previous
import jax
import jax.numpy as jnp
from jax.experimental import pallas as pl
from jax.experimental.pallas import tpu as pltpu

B, H, KV, DH = 16, 16, 4096, 128
BH = B * H


def _copy_patch_kernel(idx_ref, cur_ref, cache_ref, out_ref):
    out_ref[...] = cache_ref[...]
    kv_blk = out_ref.shape[1]
    j = pl.program_id(1)
    slot = idx_ref[0] - 1
    off = slot - j * kv_blk

    @pl.when((off >= 0) & (off < kv_blk))
    def _():
        out_ref[:, pl.ds(off, 1), :] = cur_ref[...]


def kernel(cur, dim, idx, cache):
    del dim
    cache3 = cache.reshape(BH, KV, DH)
    cur3 = cur.reshape(BH, 1, DH)

    bh_blk = min(4, BH)
    kv_blk = min(4096, KV)
    grid = (BH // bh_blk, KV // kv_blk)

    out = pl.pallas_call(
        _copy_patch_kernel,
        grid=grid,
        in_specs=[
            pl.BlockSpec(memory_space=pltpu.SMEM),
            pl.BlockSpec((bh_blk, 1, DH), lambda i, j: (i, 0, 0)),
            pl.BlockSpec((bh_blk, kv_blk, DH), lambda i, j: (i, j, 0)),
        ],
        out_specs=pl.BlockSpec((bh_blk, kv_blk, DH), lambda i, j: (i, j, 0)),
        out_shape=jax.ShapeDtypeStruct((BH, KV, DH), cache.dtype),
        compiler_params=pltpu.CompilerParams(
            dimension_semantics=("arbitrary", "arbitrary"),
            vmem_limit_bytes=63 * 1024 * 1024,
        ),
    )(idx, cur3, cache3)
    return out.reshape(B, H, KV, DH)
